# baseline (speedup 1.0000x reference)
import jax
import jax.numpy as jnp
from jax import lax
from jax.experimental import pallas as pl
from jax.experimental.pallas import tpu as pltpu
from jax.experimental.pallas import tpu_sc as plsc

H_BEV, W_BEV = 256, 256
X_MIN, X_MAX, Y_MIN, Y_MAX = -51.2, 51.2, -51.2, 51.2
NBIN = H_BEV * W_BEV
DUMP = NBIN
ACC = NBIN + 16

T_, C_, CR, HP, WP = 2, 6, 64, 64, 176
HD, WD = 32, 88
P_IMG = HP * WP
NS = 16
ROWS_PER_TILE = HP // NS
P_TILE = ROWS_PER_TILE * WP
NVEC = P_IMG // 16
CH_PER_TILE = CR // NS
NPAR = 12
PPAD = 16


def _bf16r(x):
    u = plsc.bitcast(x, jnp.uint32)
    r = u + jnp.uint32(0x7FFF) + (lax.shift_right_logical(u, jnp.uint32(16))
                                  & jnp.uint32(1))
    return plsc.bitcast(r & jnp.uint32(0xFFFF0000), jnp.float32)


def _splat_body(feats_hbm, depths_hbm, params_hbm, out_hbm,
                acc, iob, fbuf, invbuf, dbuf, pbuf,
                idx_sp, cnt_sp, inv_sp, semA, semB):
    c = lax.axis_index("c")
    s = lax.axis_index("s")
    t = c
    iota = lax.iota(jnp.int32, 16)
    inv_rx = jnp.float32(W_BEV / (X_MAX - X_MIN))
    inv_ry = jnp.float32(H_BEV / (Y_MAX - Y_MIN))

    def al8(i):
        return pl.multiple_of(i, 8)

    def cam_body(cam, _):
        n = t * C_ + cam
        pltpu.sync_copy(params_hbm.at[pl.ds(al8(n * NPAR * PPAD), NPAR * PPAD)],
                        pbuf)
        pltpu.sync_copy(
            depths_hbm.at[pl.ds(al8(n * HD * WD + s * 2 * WD), 2 * WD)], dbuf)
        ifx, cxv = pbuf[pl.ds(0, 16)], pbuf[pl.ds(16, 16)]
        ify, cyv = pbuf[pl.ds(32, 16)], pbuf[pl.ds(48, 16)]
        r00, r01 = pbuf[pl.ds(64, 16)], pbuf[pl.ds(80, 16)]
        r02, r03 = pbuf[pl.ds(96, 16)], pbuf[pl.ds(112, 16)]
        r10, r11 = pbuf[pl.ds(128, 16)], pbuf[pl.ds(144, 16)]
        r12, r13 = pbuf[pl.ds(160, 16)], pbuf[pl.ds(176, 16)]

        def row_body(r, _):
            v = ROWS_PER_TILE * s + r
            v_f = v.astype(jnp.float32)
            dvbase = jnp.broadcast_to(lax.shift_right_logical(r, 1) * WD, (16,))
            for j in range(WP // 16):
                u = iota + (16 * j)
                du = lax.shift_right_logical(u, 1)
                d = plsc.load_gather(dbuf, [dvbase + du])
                u_f = u.astype(jnp.float32)
                xc = _bf16r((u_f - cxv) * ifx * d)
                yc = _bf16r((v_f - cyv) * ify * d)
                db = _bf16r(d)
                x = r00 * xc + r01 * yc + r02 * db + r03
                y = r10 * xc + r11 * yc + r12 * db + r13
                valid = ((d > 0.0) & (x >= X_MIN) & (x < X_MAX)
                         & (y >= Y_MIN) & (y < Y_MAX))
                ix = jnp.clip((x - X_MIN) * inv_rx, 0.0, 255.0).astype(jnp.int32)
                iy = jnp.clip((y - Y_MIN) * inv_ry, 0.0, 255.0).astype(jnp.int32)
                lin = jnp.where(valid, iy * W_BEV + ix, DUMP)
                iob[pl.ds(r * WP + j * 16, 16)] = lin
            return 0

        lax.fori_loop(0, ROWS_PER_TILE, row_body, 0)
        pltpu.sync_copy(iob.at[pl.ds(0, P_TILE)],
                        idx_sp.at[pl.ds(al8(cam * P_IMG + P_TILE * s), P_TILE)])
        return 0

    lax.fori_loop(0, C_, cam_body, 0)
    plsc.subcore_barrier()

    @pl.when(s < C_)
    def _():
        def zb(i, _):
            acc[pl.ds(i * 16, 16)] = jnp.zeros((16,), jnp.float32)
            return 0
        lax.fori_loop(0, NBIN // 16, zb, 0, unroll=8)
        pltpu.sync_copy(idx_sp.at[pl.ds(al8(s * P_IMG), P_IMG)],
                        iob.at[pl.ds(0, P_IMG)])
        ones = jnp.full((16,), 1.0, jnp.float32)

        def cb(i, _):
            iv = iob[pl.ds(i * 16, 16)]
            plsc.addupdate_scatter(acc, [iv], ones, mask=iv < DUMP)
            return 0
        lax.fori_loop(0, NVEC, cb, 0, unroll=4)
        pltpu.sync_copy(acc.at[pl.ds(0, NBIN)],
                        cnt_sp.at[pl.ds(al8(s * NBIN), NBIN)])

    plsc.subcore_barrier()

    CHUNK = NBIN // NS
    for k in range(C_):
        pltpu.sync_copy(cnt_sp.at[pl.ds(al8(k * NBIN + s * CHUNK), CHUNK)],
                        acc.at[pl.ds(k * CHUNK, CHUNK)])

    def inv_body(i, _):
        den = jnp.full((16,), 0.0, jnp.float32)
        for k in range(C_):
            den = den + jnp.maximum(acc[pl.ds(k * CHUNK + i * 16, 16)], 1.0)
        invbuf[pl.ds(i * 16, 16)] = 1.0 / den
        return 0

    lax.fori_loop(0, CHUNK // 16, inv_body, 0, unroll=2)
    pltpu.sync_copy(invbuf.at[pl.ds(0, CHUNK)],
                    inv_sp.at[pl.ds(al8(s * CHUNK), CHUNK)])
    plsc.subcore_barrier()

    CHQ = P_IMG // 4
    NCHK = 4 * C_

    for p in range(CH_PER_TILE):
        ch = s * CH_PER_TILE + p

        def issue(k, bank):
            cam, q = divmod(k, 4)
            n = t * C_ + cam
            return pltpu.async_copy(
                feats_hbm.at[pl.ds(al8((n * CR + ch) * P_IMG + q * CHQ), CHQ)],
                fbuf.at[pl.ds(bank * CHQ, CHQ)], semA)

        def issue_idx(k, bank):
            cam, q = divmod(k, 4)
            return pltpu.async_copy(
                idx_sp.at[pl.ds(al8(cam * P_IMG + q * CHQ), CHQ)],
                iob.at[pl.ds(bank * CHQ, CHQ)], semB)

        pend = issue(0, 0)
        pend_i = issue_idx(0, 0)

        def zb(i, _):
            acc[pl.ds(i * 16, 16)] = jnp.zeros((16,), jnp.float32)
            return 0
        lax.fori_loop(0, NBIN // 16, zb, 0, unroll=8)

        for k in range(NCHK):
            bank = k % 2
            pend_i.wait()
            pend.wait()
            if k + 1 < NCHK:
                pend = issue(k + 1, 1 - bank)
                pend_i = issue_idx(k + 1, 1 - bank)

            def sb(i, _):
                iv = iob[pl.ds(bank * CHQ + i * 16, 16)]
                fv = fbuf[pl.ds(bank * CHQ + i * 16, 16)]
                plsc.addupdate_scatter(acc, [iv], fv, mask=iv < DUMP)
                return 0
            lax.fori_loop(0, CHQ // 16, sb, 0, unroll=4)

        CHO = NBIN // 16
        dout = None
        dinv = pltpu.async_copy(inv_sp.at[pl.ds(al8(0), CHO)],
                                invbuf.at[pl.ds(0, CHO)], semA)
        for k2 in range(16):
            bank = k2 % 2
            dinv.wait()
            if k2 + 1 < 16:
                dinv = pltpu.async_copy(
                    inv_sp.at[pl.ds(al8((k2 + 1) * CHO), CHO)],
                    invbuf.at[pl.ds((1 - bank) * CHO, CHO)], semA)

            def mb(i, _):
                off = k2 * CHO + i * 16
                acc[pl.ds(off, 16)] = (acc[pl.ds(off, 16)]
                                       * invbuf[pl.ds(bank * CHO + i * 16, 16)])
                return 0
            lax.fori_loop(0, CHO // 16, mb, 0, unroll=4)
            if dout is not None:
                dout.wait()
            dout = pltpu.async_copy(
                acc.at[pl.ds(k2 * CHO, CHO)],
                out_hbm.at[pl.ds(al8((t * CR + ch) * NBIN + k2 * CHO),
                                 CHO)],
                semB)
        dout.wait()


@jax.jit
def _splat(feats_flat, depths_flat, params_flat):
    mesh = plsc.VectorSubcoreMesh(core_axis_name="c", subcore_axis_name="s")
    run = pl.kernel(
        _splat_body,
        out_type=jax.ShapeDtypeStruct((T_ * CR * NBIN,), jnp.float32),
        mesh=mesh,
        scratch_types=[
            pltpu.VMEM((ACC,), jnp.float32),
            pltpu.VMEM((P_IMG,), jnp.int32),
            pltpu.VMEM((P_IMG // 2,), jnp.float32),
            pltpu.VMEM((NBIN // 8,), jnp.float32),
            pltpu.VMEM((2 * WD,), jnp.float32),
            pltpu.VMEM((NPAR * PPAD,), jnp.float32),
            pltpu.VMEM_SHARED((C_ * P_IMG,), jnp.int32),
            pltpu.VMEM_SHARED((C_ * NBIN,), jnp.float32),
            pltpu.VMEM_SHARED((NBIN,), jnp.float32),
            pltpu.SemaphoreType.DMA,
            pltpu.SemaphoreType.DMA,
        ],
        compiler_params=pltpu.CompilerParams(needs_layout_passes=False),
        name="lift_splat_sc",
    )
    return run(feats_flat, depths_flat, params_flat)


def kernel(feats, depths, K_scaled, T_cam_from_ego, H, W):
    B, T, C, Cr, Hp, Wp = feats.shape
    N = B * T * C
    Sy = Hp / jnp.asarray(H, jnp.float32)
    Sx = Wp / jnp.asarray(W, jnp.float32)
    fx = K_scaled[..., 0, 0] * Sx
    fy = K_scaled[..., 1, 1] * Sy
    cx = K_scaled[..., 0, 2] * Sx
    cy = K_scaled[..., 1, 2] * Sy
    R = jnp.linalg.inv(T_cam_from_ego)[..., 0:3, 0:4]
    Rb = R.astype(jnp.bfloat16).astype(jnp.float32)
    params = jnp.stack(
        [1.0 / fx, cx, 1.0 / fy, cy,
         Rb[..., 0, 0], Rb[..., 0, 1], Rb[..., 0, 2], Rb[..., 0, 3],
         Rb[..., 1, 0], Rb[..., 1, 1], Rb[..., 1, 2], Rb[..., 1, 3]],
        axis=-1).reshape(N, NPAR).astype(jnp.float32)
    params = jnp.broadcast_to(params[:, :, None], (N, NPAR, PPAD)).reshape(-1)
    feats_flat = feats.reshape(-1)
    depths_flat = depths.astype(jnp.float32).reshape(-1)
    out = _splat(feats_flat, depths_flat, params)
    return out.reshape(B, T, Cr, H_BEV, W_BEV)


# speedup vs baseline: 1.1286x; 1.1286x over previous
import jax
import jax.numpy as jnp
from jax import lax
from jax.experimental import pallas as pl
from jax.experimental.pallas import tpu as pltpu
from jax.experimental.pallas import tpu_sc as plsc

H_BEV, W_BEV = 256, 256
X_MIN, X_MAX, Y_MIN, Y_MAX = -51.2, 51.2, -51.2, 51.2
NBIN = H_BEV * W_BEV
DUMP = NBIN
ACC = NBIN + 16

T_, C_, CR, HP, WP = 2, 6, 64, 64, 176
HD, WD = 32, 88
P_IMG = HP * WP
NS = 16
ROWS_PER_TILE = HP // NS
P_TILE = ROWS_PER_TILE * WP
NVEC = P_IMG // 16
CH_PER_TILE = CR // NS
NPAR = 12
PPAD = 16


def _bf16r(x):
    u = plsc.bitcast(x, jnp.uint32)
    r = u + jnp.uint32(0x7FFF) + (lax.shift_right_logical(u, jnp.uint32(16))
                                  & jnp.uint32(1))
    return plsc.bitcast(r & jnp.uint32(0xFFFF0000), jnp.float32)


def _splat_body(feats_hbm, depths_hbm, params_hbm, out_hbm,
                acc, iob, fbuf, invbuf, dbuf, pbuf,
                idx_sp, cnt_sp, inv_sp, semA, semB):
    c = lax.axis_index("c")
    s = lax.axis_index("s")
    t = c
    iota = lax.iota(jnp.int32, 16)
    inv_rx = jnp.float32(W_BEV / (X_MAX - X_MIN))
    inv_ry = jnp.float32(H_BEV / (Y_MAX - Y_MIN))

    def al8(i):
        return pl.multiple_of(i, 8)

    def cam_body(cam, _):
        n = t * C_ + cam
        pltpu.sync_copy(params_hbm.at[pl.ds(al8(n * NPAR * PPAD), NPAR * PPAD)],
                        pbuf)
        pltpu.sync_copy(
            depths_hbm.at[pl.ds(al8(n * HD * WD + s * 2 * WD), 2 * WD)], dbuf)
        ifx, cxv = pbuf[pl.ds(0, 16)], pbuf[pl.ds(16, 16)]
        ify, cyv = pbuf[pl.ds(32, 16)], pbuf[pl.ds(48, 16)]
        r00, r01 = pbuf[pl.ds(64, 16)], pbuf[pl.ds(80, 16)]
        r02, r03 = pbuf[pl.ds(96, 16)], pbuf[pl.ds(112, 16)]
        r10, r11 = pbuf[pl.ds(128, 16)], pbuf[pl.ds(144, 16)]
        r12, r13 = pbuf[pl.ds(160, 16)], pbuf[pl.ds(176, 16)]

        def row_body(r, _):
            v = ROWS_PER_TILE * s + r
            v_f = v.astype(jnp.float32)
            dvbase = jnp.broadcast_to(lax.shift_right_logical(r, 1) * WD, (16,))
            for j in range(WP // 16):
                u = iota + (16 * j)
                du = lax.shift_right_logical(u, 1)
                d = plsc.load_gather(dbuf, [dvbase + du])
                u_f = u.astype(jnp.float32)
                xc = _bf16r((u_f - cxv) * ifx * d)
                yc = _bf16r((v_f - cyv) * ify * d)
                db = _bf16r(d)
                x = r00 * xc + r01 * yc + r02 * db + r03
                y = r10 * xc + r11 * yc + r12 * db + r13
                valid = ((d > 0.0) & (x >= X_MIN) & (x < X_MAX)
                         & (y >= Y_MIN) & (y < Y_MAX))
                ix = jnp.clip((x - X_MIN) * inv_rx, 0.0, 255.0).astype(jnp.int32)
                iy = jnp.clip((y - Y_MIN) * inv_ry, 0.0, 255.0).astype(jnp.int32)
                lin = jnp.where(valid, iy * W_BEV + ix, DUMP)
                iob[pl.ds(r * WP + j * 16, 16)] = lin
            return 0

        lax.fori_loop(0, ROWS_PER_TILE, row_body, 0)
        pltpu.sync_copy(iob.at[pl.ds(0, P_TILE)],
                        idx_sp.at[pl.ds(al8(cam * P_IMG + P_TILE * s), P_TILE)])
        return 0

    lax.fori_loop(0, C_, cam_body, 0)
    plsc.subcore_barrier()

    @pl.when(s < C_)
    def _():
        def zb(i, _):
            acc[pl.ds(i * 16, 16)] = jnp.zeros((16,), jnp.float32)
            return 0
        lax.fori_loop(0, NBIN // 16, zb, 0, unroll=8)
        pltpu.sync_copy(idx_sp.at[pl.ds(al8(s * P_IMG), P_IMG)],
                        iob.at[pl.ds(0, P_IMG)])
        ones = jnp.full((16,), 1.0, jnp.float32)

        def cb(i, _):
            iv = iob[pl.ds(i * 16, 16)]
            plsc.addupdate_scatter(acc, [iv], ones, mask=iv < DUMP)
            return 0
        lax.fori_loop(0, NVEC, cb, 0, unroll=4)
        pltpu.sync_copy(acc.at[pl.ds(0, NBIN)],
                        cnt_sp.at[pl.ds(al8(s * NBIN), NBIN)])

    plsc.subcore_barrier()

    CHUNK = NBIN // NS
    for k in range(C_):
        pltpu.sync_copy(cnt_sp.at[pl.ds(al8(k * NBIN + s * CHUNK), CHUNK)],
                        acc.at[pl.ds(k * CHUNK, CHUNK)])

    def inv_body(i, _):
        den = jnp.full((16,), 0.0, jnp.float32)
        for k in range(C_):
            den = den + jnp.maximum(acc[pl.ds(k * CHUNK + i * 16, 16)], 1.0)
        invbuf[pl.ds(i * 16, 16)] = 1.0 / den
        return 0

    lax.fori_loop(0, CHUNK // 16, inv_body, 0, unroll=2)
    pltpu.sync_copy(invbuf.at[pl.ds(0, CHUNK)],
                    inv_sp.at[pl.ds(al8(s * CHUNK), CHUNK)])
    plsc.subcore_barrier()

    CHQ = P_IMG // 4
    NCHK = 4 * C_

    for p in range(CH_PER_TILE):
        ch = s * CH_PER_TILE + p

        def issue(k, bank):
            cam, q = divmod(k, 4)
            n = t * C_ + cam
            return pltpu.async_copy(
                feats_hbm.at[pl.ds(al8((n * CR + ch) * P_IMG + q * CHQ), CHQ)],
                fbuf.at[pl.ds(bank * CHQ, CHQ)], semA)

        def issue_idx(k, bank):
            cam, q = divmod(k, 4)
            return pltpu.async_copy(
                idx_sp.at[pl.ds(al8(cam * P_IMG + q * CHQ), CHQ)],
                iob.at[pl.ds(bank * CHQ, CHQ)], semB)

        pend = issue(0, 0)
        pend_i = issue_idx(0, 0)

        def zb(i, _):
            acc[pl.ds(i * 16, 16)] = jnp.zeros((16,), jnp.float32)
            return 0
        lax.fori_loop(0, NBIN // 16, zb, 0, unroll=8)

        for k in range(NCHK):
            bank = k % 2
            pend_i.wait()
            pend.wait()
            if k + 1 < NCHK:
                pend = issue(k + 1, 1 - bank)
                pend_i = issue_idx(k + 1, 1 - bank)

            def sb(i, _):
                iv = iob[pl.ds(bank * CHQ + i * 16, 16)]
                fv = fbuf[pl.ds(bank * CHQ + i * 16, 16)]
                plsc.addupdate_scatter(acc, [iv], fv, mask=iv < DUMP)
                return 0
            lax.fori_loop(0, CHQ // 16, sb, 0, unroll=4)

        CHO = NBIN // 16
        dout = None
        dinv = pltpu.async_copy(inv_sp.at[pl.ds(al8(0), CHO)],
                                invbuf.at[pl.ds(0, CHO)], semA)
        for k2 in range(16):
            bank = k2 % 2
            dinv.wait()
            if k2 + 1 < 16:
                dinv = pltpu.async_copy(
                    inv_sp.at[pl.ds(al8((k2 + 1) * CHO), CHO)],
                    invbuf.at[pl.ds((1 - bank) * CHO, CHO)], semA)

            def mb(i, _):
                off = k2 * CHO + i * 16
                acc[pl.ds(off, 16)] = (acc[pl.ds(off, 16)]
                                       * invbuf[pl.ds(bank * CHO + i * 16, 16)])
                return 0
            lax.fori_loop(0, CHO // 16, mb, 0, unroll=8)
            if dout is not None:
                dout.wait()
            dout = pltpu.async_copy(
                acc.at[pl.ds(k2 * CHO, CHO)],
                out_hbm.at[pl.ds(al8((t * CR + ch) * NBIN + k2 * CHO),
                                 CHO)],
                semB)
        dout.wait()


@jax.jit
def _splat(feats_flat, depths_flat, params_flat):
    mesh = plsc.VectorSubcoreMesh(core_axis_name="c", subcore_axis_name="s")
    run = pl.kernel(
        _splat_body,
        out_type=jax.ShapeDtypeStruct((T_ * CR * NBIN,), jnp.float32),
        mesh=mesh,
        scratch_types=[
            pltpu.VMEM((ACC,), jnp.float32),
            pltpu.VMEM((P_IMG,), jnp.int32),
            pltpu.VMEM((P_IMG // 2,), jnp.float32),
            pltpu.VMEM((NBIN // 8,), jnp.float32),
            pltpu.VMEM((2 * WD,), jnp.float32),
            pltpu.VMEM((NPAR * PPAD,), jnp.float32),
            pltpu.VMEM_SHARED((C_ * P_IMG,), jnp.int32),
            pltpu.VMEM_SHARED((C_ * NBIN,), jnp.float32),
            pltpu.VMEM_SHARED((NBIN,), jnp.float32),
            pltpu.SemaphoreType.DMA,
            pltpu.SemaphoreType.DMA,
        ],
        compiler_params=pltpu.CompilerParams(needs_layout_passes=False),
        name="lift_splat_sc",
    )
    return run(feats_flat, depths_flat, params_flat)


def kernel(feats, depths, K_scaled, T_cam_from_ego, H, W):
    B, T, C, Cr, Hp, Wp = feats.shape
    N = B * T * C
    Sy = Hp / jnp.asarray(H, jnp.float32)
    Sx = Wp / jnp.asarray(W, jnp.float32)
    fx = K_scaled[..., 0, 0] * Sx
    fy = K_scaled[..., 1, 1] * Sy
    cx = K_scaled[..., 0, 2] * Sx
    cy = K_scaled[..., 1, 2] * Sy
    R = jnp.linalg.inv(T_cam_from_ego)[..., 0:3, 0:4]
    Rb = R.astype(jnp.bfloat16).astype(jnp.float32)
    params = jnp.stack(
        [1.0 / fx, cx, 1.0 / fy, cy,
         Rb[..., 0, 0], Rb[..., 0, 1], Rb[..., 0, 2], Rb[..., 0, 3],
         Rb[..., 1, 0], Rb[..., 1, 1], Rb[..., 1, 2], Rb[..., 1, 3]],
        axis=-1).reshape(N, NPAR).astype(jnp.float32)
    params = jnp.broadcast_to(params[:, :, None], (N, NPAR, PPAD)).reshape(-1)
    feats_flat = feats.reshape(-1)
    depths_flat = depths.astype(jnp.float32).reshape(-1)
    out = _splat(feats_flat, depths_flat, params)
    return out.reshape(B, T, Cr, H_BEV, W_BEV)
